# baseline (speedup 1.0000x reference)
import jax
import jax.numpy as jnp
from jax import lax
from jax.experimental import pallas as pl
from jax.experimental.pallas import tpu as pltpu

_TOPK = 64
_ENC_ROWS = 256
_ENC_CHUNK = 1024
_DEC_ROWS = 64


def _abs_bits(zc):
    return lax.bitcast_convert_type(zc, jnp.int32) & jnp.int32(0x7FFFFFFF)


def _encode_topk_kernel(x_ref, we_ref, be_ref, zs_ref, a16_ref):
    j = pl.program_id(1)
    n_chunks = pl.num_programs(1)
    b = x_ref.shape[0]
    lb = we_ref.shape[1]
    d_lat = zs_ref.shape[1]

    z = jnp.dot(x_ref[...], we_ref[...], preferred_element_type=jnp.float32)
    z = z + be_ref[...]
    zs_ref[:, pl.ds(j * lb, lb)] = z
    a16_ref[:, pl.ds(j * lb, lb)] = (
        (_abs_bits(z) >> 15) - jnp.int32(32768)
    ).astype(jnp.int16)

    @pl.when(j == n_chunks - 1)
    def _select():
        def count16_ge(cand16):
            candb = (cand16 - jnp.int32(32768)).astype(jnp.int16)

            def body(c, acc):
                a = a16_ref[:, pl.ds(c * lb, lb)]
                h = (a >= candb).astype(jnp.bfloat16)
                for s in range(0, lb, 256):
                    part = jnp.sum(
                        h[:, s : s + 256],
                        axis=1,
                        keepdims=True,
                        dtype=jnp.bfloat16,
                    )
                    acc = acc + part.astype(jnp.float32)
                return acc

            cnt16 = lax.fori_loop(0, n_chunks, body, jnp.zeros((b, 1), jnp.float32))
            return cnt16.astype(jnp.int32)

        def count32_ge(cand):
            def body(c, acc):
                a = _abs_bits(zs_ref[:, pl.ds(c * lb, lb)])
                return acc + jnp.sum(
                    (a >= cand).astype(jnp.int32), axis=1, keepdims=True
                )

            return lax.fori_loop(0, n_chunks, body, jnp.zeros((b, 1), jnp.int32))

        def val_bit16(i, p16):
            cand16 = p16 | (jnp.int32(1) << (15 - i))
            return jnp.where(count16_ge(cand16) >= _TOPK, cand16, p16)

        p16 = lax.fori_loop(0, 16, val_bit16, jnp.zeros((b, 1), jnp.int32))

        def val_bit32(i, p):
            cand = p | (jnp.int32(1) << (14 - i))
            return jnp.where(count32_ge(cand) >= _TOPK, cand, p)

        p = lax.fori_loop(0, 15, val_bit32, p16 << 15)

        def cnt_body(c, accs):
            acc_ge, acc_gt = accs
            a = _abs_bits(zs_ref[:, pl.ds(c * lb, lb)])
            acc_ge = acc_ge + jnp.sum((a >= p).astype(jnp.int32), axis=1, keepdims=True)
            acc_gt = acc_gt + jnp.sum((a > p).astype(jnp.int32), axis=1, keepdims=True)
            return acc_ge, acc_gt

        z0 = jnp.zeros((b, 1), jnp.int32)
        cnt_ge, cnt_gt = lax.fori_loop(0, n_chunks, cnt_body, (z0, z0))
        need = _TOPK - cnt_gt

        def idx_search(_):
            nbits = max(1, (d_lat - 1).bit_length())

            def idx_bit(i, t):
                test = t | (jnp.int32(1) << (nbits - 1 - i))

                def body(c, acc):
                    a = _abs_bits(zs_ref[:, pl.ds(c * lb, lb)])
                    idx = c * lb + lax.broadcasted_iota(jnp.int32, (b, lb), 1)
                    hit = (a == p) & (idx < test)
                    return acc + jnp.sum(hit.astype(jnp.int32), axis=1, keepdims=True)

                cnt = lax.fori_loop(0, n_chunks, body, jnp.zeros((b, 1), jnp.int32))
                return jnp.where(cnt < need, test, t)

            return lax.fori_loop(0, nbits, idx_bit, jnp.zeros((b, 1), jnp.int32))

        t = lax.cond(
            jnp.any(cnt_ge > _TOPK),
            idx_search,
            lambda _: jnp.full((b, 1), d_lat, jnp.int32),
            operand=None,
        )

        def mask_chunk(c, _):
            zc = zs_ref[:, pl.ds(c * lb, lb)]
            a = _abs_bits(zc)
            idx = c * lb + lax.broadcasted_iota(jnp.int32, (b, lb), 1)
            keep = (a > p) | ((a == p) & (idx <= t))
            zs_ref[:, pl.ds(c * lb, lb)] = jnp.where(keep, zc, 0.0)
            return 0

        lax.fori_loop(0, n_chunks, mask_chunk, 0)


def _decode_kernel(zs_ref, wd_ref, bd_ref, recon_ref):
    recon_ref[...] = (
        jnp.dot(zs_ref[...], wd_ref[...], preferred_element_type=jnp.float32)
        + bd_ref[...]
    )


def kernel(x, W_enc, b_enc, W_dec, b_dec):
    n_tok, d_in = x.shape
    d_lat = W_enc.shape[1]
    be2 = b_enc.reshape(1, d_lat)
    bd2 = b_dec.reshape(1, d_in)

    b1 = min(_ENC_ROWS, n_tok)
    lb = min(_ENC_CHUNK, d_lat)
    zs = pl.pallas_call(
        _encode_topk_kernel,
        grid=(n_tok // b1, d_lat // lb),
        in_specs=[
            pl.BlockSpec((b1, d_in), lambda i, j: (i, 0)),
            pl.BlockSpec((d_in, lb), lambda i, j: (0, j)),
            pl.BlockSpec((1, lb), lambda i, j: (0, j)),
        ],
        out_specs=pl.BlockSpec((b1, d_lat), lambda i, j: (i, 0)),
        out_shape=jax.ShapeDtypeStruct((n_tok, d_lat), jnp.float32),
        scratch_shapes=[pltpu.VMEM((b1, d_lat), jnp.int16)],
        compiler_params=pltpu.CompilerParams(
            dimension_semantics=("arbitrary", "arbitrary"),
        ),
    )(x, W_enc, be2)

    b2 = min(_DEC_ROWS, n_tok)
    recon = pl.pallas_call(
        _decode_kernel,
        grid=(n_tok // b2,),
        in_specs=[
            pl.BlockSpec((b2, d_lat), lambda i: (i, 0)),
            pl.BlockSpec((d_lat, d_in), lambda i: (0, 0)),
            pl.BlockSpec((1, d_in), lambda i: (0, 0)),
        ],
        out_specs=pl.BlockSpec((b2, d_in), lambda i: (i, 0)),
        out_shape=jax.ShapeDtypeStruct((n_tok, d_in), jnp.float32),
        compiler_params=pltpu.CompilerParams(
            dimension_semantics=("arbitrary",),
        ),
    )(zs, W_dec, bd2)
    return (recon, zs)


# speedup vs baseline: 1.1299x; 1.1299x over previous
import jax
import jax.numpy as jnp
from jax import lax
from jax.experimental import pallas as pl
from jax.experimental.pallas import tpu as pltpu

_TOPK = 64
_ENC_ROWS = 256
_ENC_CHUNK = 1024
_DEC_ROWS = 64


def _abs_bits(zc):
    return lax.bitcast_convert_type(zc, jnp.int32) & jnp.int32(0x7FFFFFFF)


def _encode_topk_kernel(x_ref, we_ref, be_ref, zs_ref, ai_ref):
    j = pl.program_id(1)
    n_chunks = pl.num_programs(1)
    b = x_ref.shape[0]
    lb = we_ref.shape[1]
    d_lat = zs_ref.shape[1]

    z = jnp.dot(x_ref[...], we_ref[...], preferred_element_type=jnp.float32)
    z = z + be_ref[...]
    zs_ref[:, pl.ds(j * lb, lb)] = z
    ai_ref[:, pl.ds(j * lb, lb)] = _abs_bits(z)

    @pl.when(j == n_chunks - 1)
    def _select():
        def count_ge(cand):
            def body(c, acc):
                a = ai_ref[:, pl.ds(c * lb, lb)]
                return acc + jnp.sum(
                    (a >= cand).astype(jnp.int32), axis=1, keepdims=True
                )

            return lax.fori_loop(0, n_chunks, body, jnp.zeros((b, 1), jnp.int32))

        def val_bit(i, p):
            cand = p | (jnp.int32(1) << (30 - i))
            return jnp.where(count_ge(cand) >= _TOPK, cand, p)

        p = lax.fori_loop(0, 31, val_bit, jnp.zeros((b, 1), jnp.int32))

        def cnt_body(c, accs):
            acc_ge, acc_gt = accs
            a = ai_ref[:, pl.ds(c * lb, lb)]
            acc_ge = acc_ge + jnp.sum((a >= p).astype(jnp.int32), axis=1, keepdims=True)
            acc_gt = acc_gt + jnp.sum((a > p).astype(jnp.int32), axis=1, keepdims=True)
            return acc_ge, acc_gt

        z0 = jnp.zeros((b, 1), jnp.int32)
        cnt_ge, cnt_gt = lax.fori_loop(0, n_chunks, cnt_body, (z0, z0))
        need = _TOPK - cnt_gt

        def idx_search(_):
            nbits = max(1, (d_lat - 1).bit_length())

            def idx_bit(i, t):
                test = t | (jnp.int32(1) << (nbits - 1 - i))

                def body(c, acc):
                    a = ai_ref[:, pl.ds(c * lb, lb)]
                    idx = c * lb + lax.broadcasted_iota(jnp.int32, (b, lb), 1)
                    hit = (a == p) & (idx < test)
                    return acc + jnp.sum(hit.astype(jnp.int32), axis=1, keepdims=True)

                cnt = lax.fori_loop(0, n_chunks, body, jnp.zeros((b, 1), jnp.int32))
                return jnp.where(cnt < need, test, t)

            return lax.fori_loop(0, nbits, idx_bit, jnp.zeros((b, 1), jnp.int32))

        t = lax.cond(
            jnp.any(cnt_ge > _TOPK),
            idx_search,
            lambda _: jnp.full((b, 1), d_lat, jnp.int32),
            operand=None,
        )

        def mask_chunk(c, _):
            zc = zs_ref[:, pl.ds(c * lb, lb)]
            a = _abs_bits(zc)
            idx = c * lb + lax.broadcasted_iota(jnp.int32, (b, lb), 1)
            keep = (a > p) | ((a == p) & (idx <= t))
            zs_ref[:, pl.ds(c * lb, lb)] = jnp.where(keep, zc, 0.0)
            return 0

        lax.fori_loop(0, n_chunks, mask_chunk, 0)


def _decode_kernel(zs_ref, wd_ref, bd_ref, recon_ref):
    recon_ref[...] = (
        jnp.dot(zs_ref[...], wd_ref[...], preferred_element_type=jnp.float32)
        + bd_ref[...]
    )


def kernel(x, W_enc, b_enc, W_dec, b_dec):
    n_tok, d_in = x.shape
    d_lat = W_enc.shape[1]
    be2 = b_enc.reshape(1, d_lat)
    bd2 = b_dec.reshape(1, d_in)

    b1 = min(_ENC_ROWS, n_tok)
    lb = min(_ENC_CHUNK, d_lat)
    zs = pl.pallas_call(
        _encode_topk_kernel,
        grid=(n_tok // b1, d_lat // lb),
        in_specs=[
            pl.BlockSpec((b1, d_in), lambda i, j: (i, 0)),
            pl.BlockSpec((d_in, lb), lambda i, j: (0, j)),
            pl.BlockSpec((1, lb), lambda i, j: (0, j)),
        ],
        out_specs=pl.BlockSpec((b1, d_lat), lambda i, j: (i, 0)),
        out_shape=jax.ShapeDtypeStruct((n_tok, d_lat), jnp.float32),
        scratch_shapes=[pltpu.VMEM((b1, d_lat), jnp.int32)],
        compiler_params=pltpu.CompilerParams(
            dimension_semantics=("arbitrary", "arbitrary"),
        ),
    )(x, W_enc, be2)

    b2 = min(_DEC_ROWS, n_tok)
    recon = pl.pallas_call(
        _decode_kernel,
        grid=(n_tok // b2,),
        in_specs=[
            pl.BlockSpec((b2, d_lat), lambda i: (i, 0)),
            pl.BlockSpec((d_lat, d_in), lambda i: (0, 0)),
            pl.BlockSpec((1, d_in), lambda i: (0, 0)),
        ],
        out_specs=pl.BlockSpec((b2, d_in), lambda i: (i, 0)),
        out_shape=jax.ShapeDtypeStruct((n_tok, d_in), jnp.float32),
        compiler_params=pltpu.CompilerParams(
            dimension_semantics=("arbitrary",),
        ),
    )(zs, W_dec, bd2)
    return (recon, zs)


# speedup vs baseline: 1.7015x; 1.5059x over previous
import jax
import jax.numpy as jnp
from jax import lax
from jax.experimental import pallas as pl
from jax.experimental.pallas import tpu as pltpu

_TOPK = 64
_ENC_ROWS = 256
_ENC_CHUNK = 1024
_DEC_ROWS = 64


def _abs_bits(zc):
    return lax.bitcast_convert_type(zc, jnp.int32) & jnp.int32(0x7FFFFFFF)


def _encode_topk_kernel(x_ref, we_ref, be_ref, zs_ref, ai_ref):
    j = pl.program_id(1)
    n_chunks = pl.num_programs(1)
    b = x_ref.shape[0]
    lb = we_ref.shape[1]
    d_lat = zs_ref.shape[1]

    z = jnp.dot(x_ref[...], we_ref[...], preferred_element_type=jnp.float32)
    z = z + be_ref[...]
    zs_ref[:, pl.ds(j * lb, lb)] = z
    ai_ref[:, pl.ds(j * lb, lb)] = _abs_bits(z)

    @pl.when(j == n_chunks - 1)
    def _select():
        def count_ge(cand):
            return jnp.sum(
                (ai_ref[...] >= cand).astype(jnp.int32), axis=1, keepdims=True
            )

        def val_bit(i, p):
            cand = p | (jnp.int32(1) << (30 - i))
            return jnp.where(count_ge(cand) >= _TOPK, cand, p)

        p = lax.fori_loop(0, 31, val_bit, jnp.zeros((b, 1), jnp.int32))

        a_all = ai_ref[...]
        cnt_ge = jnp.sum((a_all >= p).astype(jnp.int32), axis=1, keepdims=True)
        cnt_gt = jnp.sum((a_all > p).astype(jnp.int32), axis=1, keepdims=True)
        need = _TOPK - cnt_gt

        def idx_search(_):
            nbits = max(1, (d_lat - 1).bit_length())

            def idx_bit(i, t):
                test = t | (jnp.int32(1) << (nbits - 1 - i))

                def body(c, acc):
                    a = ai_ref[:, pl.ds(c * lb, lb)]
                    idx = c * lb + lax.broadcasted_iota(jnp.int32, (b, lb), 1)
                    hit = (a == p) & (idx < test)
                    return acc + jnp.sum(hit.astype(jnp.int32), axis=1, keepdims=True)

                cnt = lax.fori_loop(0, n_chunks, body, jnp.zeros((b, 1), jnp.int32))
                return jnp.where(cnt < need, test, t)

            return lax.fori_loop(0, nbits, idx_bit, jnp.zeros((b, 1), jnp.int32))

        t = lax.cond(
            jnp.any(cnt_ge > _TOPK),
            idx_search,
            lambda _: jnp.full((b, 1), d_lat, jnp.int32),
            operand=None,
        )

        def mask_chunk(c, _):
            zc = zs_ref[:, pl.ds(c * lb, lb)]
            a = _abs_bits(zc)
            idx = c * lb + lax.broadcasted_iota(jnp.int32, (b, lb), 1)
            keep = (a > p) | ((a == p) & (idx <= t))
            zs_ref[:, pl.ds(c * lb, lb)] = jnp.where(keep, zc, 0.0)
            return 0

        lax.fori_loop(0, n_chunks, mask_chunk, 0)


def _decode_kernel(zs_ref, wd_ref, bd_ref, recon_ref):
    recon_ref[...] = (
        jnp.dot(zs_ref[...], wd_ref[...], preferred_element_type=jnp.float32)
        + bd_ref[...]
    )


def kernel(x, W_enc, b_enc, W_dec, b_dec):
    n_tok, d_in = x.shape
    d_lat = W_enc.shape[1]
    be2 = b_enc.reshape(1, d_lat)
    bd2 = b_dec.reshape(1, d_in)

    b1 = min(_ENC_ROWS, n_tok)
    lb = min(_ENC_CHUNK, d_lat)
    zs = pl.pallas_call(
        _encode_topk_kernel,
        grid=(n_tok // b1, d_lat // lb),
        in_specs=[
            pl.BlockSpec((b1, d_in), lambda i, j: (i, 0)),
            pl.BlockSpec((d_in, lb), lambda i, j: (0, j)),
            pl.BlockSpec((1, lb), lambda i, j: (0, j)),
        ],
        out_specs=pl.BlockSpec((b1, d_lat), lambda i, j: (i, 0)),
        out_shape=jax.ShapeDtypeStruct((n_tok, d_lat), jnp.float32),
        scratch_shapes=[pltpu.VMEM((b1, d_lat), jnp.int32)],
        compiler_params=pltpu.CompilerParams(
            dimension_semantics=("arbitrary", "arbitrary"),
        ),
    )(x, W_enc, be2)

    b2 = min(_DEC_ROWS, n_tok)
    recon = pl.pallas_call(
        _decode_kernel,
        grid=(n_tok // b2,),
        in_specs=[
            pl.BlockSpec((b2, d_lat), lambda i: (i, 0)),
            pl.BlockSpec((d_lat, d_in), lambda i: (0, 0)),
            pl.BlockSpec((1, d_in), lambda i: (0, 0)),
        ],
        out_specs=pl.BlockSpec((b2, d_in), lambda i: (i, 0)),
        out_shape=jax.ShapeDtypeStruct((n_tok, d_in), jnp.float32),
        compiler_params=pltpu.CompilerParams(
            dimension_semantics=("arbitrary",),
        ),
    )(zs, W_dec, bd2)
    return (recon, zs)


# speedup vs baseline: 1.7026x; 1.0007x over previous
import jax
import jax.numpy as jnp
from jax import lax
from jax.experimental import pallas as pl
from jax.experimental.pallas import tpu as pltpu

_TOPK = 64
_ENC_ROWS = 256
_ENC_CHUNK = 1024
_DEC_ROWS = 64


def _abs_bits(zc):
    return lax.bitcast_convert_type(zc, jnp.int32) & jnp.int32(0x7FFFFFFF)


def _encode_topk_kernel(x_ref, we_ref, be_ref, zs_ref, ai_ref):
    j = pl.program_id(1)
    n_chunks = pl.num_programs(1)
    b = x_ref.shape[0]
    lb = we_ref.shape[1]
    d_lat = zs_ref.shape[1]

    z = jnp.dot(x_ref[...], we_ref[...], preferred_element_type=jnp.float32)
    z = z + be_ref[...]
    zs_ref[:, pl.ds(j * lb, lb)] = z
    ai_ref[:, pl.ds(j * lb, lb)] = _abs_bits(z)

    @pl.when(j == n_chunks - 1)
    def _select():
        def count_ge(cand):
            return jnp.sum(
                (ai_ref[...] >= cand).astype(jnp.int32), axis=1, keepdims=True
            )

        def val_bit(i, p):
            cand = p | (jnp.int32(1) << (30 - i))
            return jnp.where(count_ge(cand) >= _TOPK, cand, p)

        p = lax.fori_loop(0, 31, val_bit, jnp.zeros((b, 1), jnp.int32))

        a_all = ai_ref[...]
        cnt_ge = jnp.sum((a_all >= p).astype(jnp.int32), axis=1, keepdims=True)
        cnt_gt = jnp.sum((a_all > p).astype(jnp.int32), axis=1, keepdims=True)
        need = _TOPK - cnt_gt

        def idx_search(_):
            nbits = max(1, (d_lat - 1).bit_length())

            def idx_bit(i, t):
                test = t | (jnp.int32(1) << (nbits - 1 - i))

                def body(c, acc):
                    a = ai_ref[:, pl.ds(c * lb, lb)]
                    idx = c * lb + lax.broadcasted_iota(jnp.int32, (b, lb), 1)
                    hit = (a == p) & (idx < test)
                    return acc + jnp.sum(hit.astype(jnp.int32), axis=1, keepdims=True)

                cnt = lax.fori_loop(0, n_chunks, body, jnp.zeros((b, 1), jnp.int32))
                return jnp.where(cnt < need, test, t)

            return lax.fori_loop(0, nbits, idx_bit, jnp.zeros((b, 1), jnp.int32))

        t = lax.cond(
            jnp.any(cnt_ge > _TOPK),
            idx_search,
            lambda _: jnp.full((b, 1), d_lat, jnp.int32),
            operand=None,
        )

        def mask_chunk(c, _):
            zc = zs_ref[:, pl.ds(c * lb, lb)]
            a = _abs_bits(zc)
            idx = c * lb + lax.broadcasted_iota(jnp.int32, (b, lb), 1)
            keep = (a > p) | ((a == p) & (idx <= t))
            zs_ref[:, pl.ds(c * lb, lb)] = jnp.where(keep, zc, 0.0)
            return 0

        lax.fori_loop(0, n_chunks, mask_chunk, 0)


def _decode_kernel(zs_ref, wd_ref, bd_ref, recon_ref):
    recon_ref[...] = (
        jnp.dot(
            zs_ref[...],
            wd_ref[...],
            preferred_element_type=jnp.float32,
            precision=lax.Precision.DEFAULT,
        )
        + bd_ref[...]
    )


def kernel(x, W_enc, b_enc, W_dec, b_dec):
    n_tok, d_in = x.shape
    d_lat = W_enc.shape[1]
    be2 = b_enc.reshape(1, d_lat)
    bd2 = b_dec.reshape(1, d_in)

    b1 = min(_ENC_ROWS, n_tok)
    lb = min(_ENC_CHUNK, d_lat)
    zs = pl.pallas_call(
        _encode_topk_kernel,
        grid=(n_tok // b1, d_lat // lb),
        in_specs=[
            pl.BlockSpec((b1, d_in), lambda i, j: (i, 0)),
            pl.BlockSpec((d_in, lb), lambda i, j: (0, j)),
            pl.BlockSpec((1, lb), lambda i, j: (0, j)),
        ],
        out_specs=pl.BlockSpec((b1, d_lat), lambda i, j: (i, 0)),
        out_shape=jax.ShapeDtypeStruct((n_tok, d_lat), jnp.float32),
        scratch_shapes=[pltpu.VMEM((b1, d_lat), jnp.int32)],
        compiler_params=pltpu.CompilerParams(
            dimension_semantics=("arbitrary", "arbitrary"),
        ),
    )(x, W_enc, be2)

    b2 = min(_DEC_ROWS, n_tok)
    recon = pl.pallas_call(
        _decode_kernel,
        grid=(n_tok // b2,),
        in_specs=[
            pl.BlockSpec((b2, d_lat), lambda i: (i, 0)),
            pl.BlockSpec((d_lat, d_in), lambda i: (0, 0)),
            pl.BlockSpec((1, d_in), lambda i: (0, 0)),
        ],
        out_specs=pl.BlockSpec((b2, d_in), lambda i: (i, 0)),
        out_shape=jax.ShapeDtypeStruct((n_tok, d_in), jnp.float32),
        compiler_params=pltpu.CompilerParams(
            dimension_semantics=("arbitrary",),
        ),
    )(zs, W_dec, bd2)
    return (recon, zs)


# speedup vs baseline: 1.7308x; 1.0166x over previous
import jax
import jax.numpy as jnp
from jax import lax
from jax.experimental import pallas as pl
from jax.experimental.pallas import tpu as pltpu

_TOPK = 64
_ENC_ROWS = 128
_ENC_CHUNK = 2048
_DEC_ROWS = 64


def _encode_topk_kernel(x_ref, we_ref, be_ref, zs_ref, a2_ref, s2_ref, p_ref):
    i = pl.program_id(0)
    j = pl.program_id(1)
    n_i = pl.num_programs(0)
    n_chunks = pl.num_programs(1)
    b = a2_ref.shape[1]
    d_lat = a2_ref.shape[2]
    lb = we_ref.shape[1]
    cur = i % 2
    prev = (i + 1) % 2

    bpu = -(-31 // n_chunks)
    rem = 31 - (n_chunks - 1) * bpu

    @pl.when(i < n_i - 1)
    def _matmul():
        z = jnp.dot(x_ref[...], we_ref[...], preferred_element_type=jnp.float32)
        z = z + be_ref[...]
        a2_ref[cur, :, pl.ds(j * lb, lb)] = jnp.abs(z)
        s2_ref[cur, :, pl.ds(j * lb, lb)] = jnp.sign(z).astype(jnp.int8)

    @pl.when(i > 0)
    def _select_slice():
        def count_ge(cand):
            candf = lax.bitcast_convert_type(cand, jnp.float32)
            a = a2_ref[prev]
            return jnp.sum((a >= candf).astype(jnp.int32), axis=1, keepdims=True)

        def val_bit(p, bit):
            bitc = jnp.maximum(bit, 0)
            cand = p | (jnp.int32(1) << bitc)
            ok = (count_ge(cand) >= _TOPK) & (bit >= 0)
            return jnp.where(ok, cand, p)

        p0 = jnp.where(j == 0, jnp.zeros((b, 1), jnp.int32), p_ref[...])

        @pl.when(j < n_chunks - 1)
        def _bits():
            p = p0
            for s in range(bpu):
                p = val_bit(p, 30 - bpu * j - s)
            p_ref[...] = p

        @pl.when(j == n_chunks - 1)
        def _epilogue():
            p = p0
            for s in range(rem):
                p = val_bit(p, jnp.int32(rem - 1 - s))
            pf = lax.bitcast_convert_type(p, jnp.float32)

            a_all = a2_ref[prev]
            cnt_ge = jnp.sum((a_all >= pf).astype(jnp.int32), axis=1, keepdims=True)
            cnt_gt = jnp.sum((a_all > pf).astype(jnp.int32), axis=1, keepdims=True)
            need = _TOPK - cnt_gt

            def idx_search(_):
                nbits = max(1, (d_lat - 1).bit_length())

                def idx_bit(s2, t):
                    test = t | (jnp.int32(1) << (nbits - 1 - s2))

                    def body(c, acc):
                        a = a2_ref[prev, :, pl.ds(c * lb, lb)]
                        idx = c * lb + lax.broadcasted_iota(jnp.int32, (b, lb), 1)
                        hit = (a == pf) & (idx < test)
                        return acc + jnp.sum(
                            hit.astype(jnp.int32), axis=1, keepdims=True
                        )

                    cnt = lax.fori_loop(
                        0, n_chunks, body, jnp.zeros((b, 1), jnp.int32)
                    )
                    return jnp.where(cnt < need, test, t)

                return lax.fori_loop(0, nbits, idx_bit, jnp.zeros((b, 1), jnp.int32))

            t = lax.cond(
                jnp.any(cnt_ge > _TOPK),
                idx_search,
                lambda _: jnp.full((b, 1), d_lat, jnp.int32),
                operand=None,
            )

            def mask_chunk(c, _):
                a = a2_ref[prev, :, pl.ds(c * lb, lb)]
                sg = s2_ref[prev, :, pl.ds(c * lb, lb)].astype(jnp.float32)
                idx = c * lb + lax.broadcasted_iota(jnp.int32, (b, lb), 1)
                keep = (a > pf) | ((a == pf) & (idx <= t))
                zs_ref[:, pl.ds(c * lb, lb)] = jnp.where(keep, a * sg, 0.0)
                return 0

            lax.fori_loop(0, n_chunks, mask_chunk, 0)


def _decode_kernel(zs_ref, wd_ref, bd_ref, recon_ref):
    recon_ref[...] = (
        jnp.dot(zs_ref[...], wd_ref[...], preferred_element_type=jnp.float32)
        + bd_ref[...]
    )


def kernel(x, W_enc, b_enc, W_dec, b_dec):
    n_tok, d_in = x.shape
    d_lat = W_enc.shape[1]
    be2 = b_enc.reshape(1, d_lat)
    bd2 = b_dec.reshape(1, d_in)

    b1 = min(_ENC_ROWS, n_tok)
    lb = min(_ENC_CHUNK, d_lat)
    n_blocks = n_tok // b1
    zs = pl.pallas_call(
        _encode_topk_kernel,
        grid=(n_blocks + 1, d_lat // lb),
        in_specs=[
            pl.BlockSpec((b1, d_in), lambda i, j: (jnp.minimum(i, n_blocks - 1), 0)),
            pl.BlockSpec((d_in, lb), lambda i, j: (0, j)),
            pl.BlockSpec((1, lb), lambda i, j: (0, j)),
        ],
        out_specs=pl.BlockSpec((b1, d_lat), lambda i, j: (jnp.maximum(i, 1) - 1, 0)),
        out_shape=jax.ShapeDtypeStruct((n_tok, d_lat), jnp.float32),
        scratch_shapes=[
            pltpu.VMEM((2, b1, d_lat), jnp.float32),
            pltpu.VMEM((2, b1, d_lat), jnp.int8),
            pltpu.VMEM((b1, 1), jnp.int32),
        ],
        compiler_params=pltpu.CompilerParams(
            dimension_semantics=("arbitrary", "arbitrary"),
        ),
    )(x, W_enc, be2)

    b2 = min(_DEC_ROWS, n_tok)
    recon = pl.pallas_call(
        _decode_kernel,
        grid=(n_tok // b2,),
        in_specs=[
            pl.BlockSpec((b2, d_lat), lambda i: (i, 0)),
            pl.BlockSpec((d_lat, d_in), lambda i: (0, 0)),
            pl.BlockSpec((1, d_in), lambda i: (0, 0)),
        ],
        out_specs=pl.BlockSpec((b2, d_in), lambda i: (i, 0)),
        out_shape=jax.ShapeDtypeStruct((n_tok, d_in), jnp.float32),
        compiler_params=pltpu.CompilerParams(
            dimension_semantics=("arbitrary",),
        ),
    )(zs, W_dec, bd2)
    return (recon, zs)
